# Initial kernel scaffold; baseline (speedup 1.0000x reference)
import jax
import jax.numpy as jnp
from jax.experimental import pallas as pl


def kernel(efeat, nfeat, src, dst, W_e, W_s, W_d, b1, W_out, b_out, gamma, beta):
    raise NotImplementedError("write your pallas kernel here")


# speedup vs baseline: 3.5956x; 3.5956x over previous
import functools

import jax
import jax.numpy as jnp
from jax import lax
from jax.experimental import pallas as pl
from jax.experimental.pallas import tpu as pltpu
from jax.experimental.pallas import tpu_sc as plsc

N_NODES = 10000
N_EDGES = 320000
DIM = 128

_NC = 2
_NS = 16
_NW = _NC * _NS
_EPW = N_EDGES // _NW
_CHUNK = 400
_NCHUNK = _EPW // _CHUNK


def _proj_body(n_ref, ws_ref, wd_ref, a_ref, b_ref):
    n = n_ref[...]
    dn = (((1,), (1,)), ((), ()))
    a_ref[...] = lax.dot_general(n, ws_ref[...], dn,
                                 preferred_element_type=jnp.float32)
    b_ref[...] = lax.dot_general(n, wd_ref[...], dn,
                                 preferred_element_type=jnp.float32)


def _node_proj(nfeat, W_s, W_d):
    out_sd = jax.ShapeDtypeStruct((N_NODES, DIM), jnp.float32)
    return pl.pallas_call(
        _proj_body,
        out_shape=(out_sd, out_sd),
    )(nfeat, W_s, W_d)


def _gather_body(a_hbm, b_hbm, src_hbm, dst_hbm, gs_hbm, gd_hbm,
                 idx_s, idx_d, rows_s, rows_d, sem_s, sem_d):
    wid = lax.axis_index("s") * _NC + lax.axis_index("c")
    wbase = wid * _EPW

    @pl.loop(0, _NCHUNK)
    def _chunk(j):
        base = wbase + j * _CHUNK
        pltpu.sync_copy(src_hbm.at[pl.ds(base, _CHUNK)], idx_s)
        pltpu.sync_copy(dst_hbm.at[pl.ds(base, _CHUNK)], idx_d)
        cp_s = pltpu.async_copy(a_hbm.at[idx_s], rows_s, sem_s)
        cp_d = pltpu.async_copy(b_hbm.at[idx_d], rows_d, sem_d)
        cp_s.wait()
        cp_d.wait()
        pltpu.sync_copy(rows_s, gs_hbm.at[pl.ds(base, _CHUNK)])
        pltpu.sync_copy(rows_d, gd_hbm.at[pl.ds(base, _CHUNK)])


def _edge_gather(a, b, src, dst):
    out_sd = jax.ShapeDtypeStruct((N_EDGES, DIM), jnp.float32)
    mesh = plsc.VectorSubcoreMesh(core_axis_name="c", subcore_axis_name="s")
    f = functools.partial(
        pl.kernel,
        out_type=(out_sd, out_sd),
        mesh=mesh,
        scratch_types=[
            pltpu.VMEM((_CHUNK,), jnp.int32),
            pltpu.VMEM((_CHUNK,), jnp.int32),
            pltpu.VMEM((_CHUNK, DIM), jnp.float32),
            pltpu.VMEM((_CHUNK, DIM), jnp.float32),
            pltpu.SemaphoreType.DMA,
            pltpu.SemaphoreType.DMA,
        ],
    )(_gather_body)
    return f(a, b, src, dst)


def _edge_body(e_ref, gs_ref, gd_ref, we_ref, b1_ref, wo_ref, bo_ref,
               g_ref, bt_ref, out_ref):
    e = e_ref[...]
    dn = (((1,), (1,)), ((), ()))
    h = lax.dot_general(e, we_ref[...], dn, preferred_element_type=jnp.float32)
    h = h + gs_ref[...] + gd_ref[...] + b1_ref[...]
    h = h * (1.0 / (1.0 + jnp.exp(-h)))
    o = lax.dot_general(h, wo_ref[...], dn, preferred_element_type=jnp.float32)
    o = o + bo_ref[...]
    mean = jnp.mean(o, axis=-1, keepdims=True)
    cen = o - mean
    var = jnp.mean(cen * cen, axis=-1, keepdims=True)
    o = cen * lax.rsqrt(var + 1e-5) * g_ref[...] + bt_ref[...]
    out_ref[...] = o + e


def _edge_mlp(efeat, gs, gd, W_e, b1, W_out, b_out, gamma, beta):
    blk = 2560
    grid = (N_EDGES // blk,)
    row_spec = pl.BlockSpec((blk, DIM), lambda i: (i, 0))
    w_spec = pl.BlockSpec((DIM, DIM), lambda i: (0, 0))
    v_spec = pl.BlockSpec((1, DIM), lambda i: (0, 0))
    return pl.pallas_call(
        _edge_body,
        grid=grid,
        in_specs=[row_spec, row_spec, row_spec, w_spec, v_spec, w_spec,
                  v_spec, v_spec, v_spec],
        out_specs=row_spec,
        out_shape=jax.ShapeDtypeStruct((N_EDGES, DIM), jnp.float32),
    )(efeat, gs, gd, W_e, b1.reshape(1, DIM), W_out, b_out.reshape(1, DIM),
      gamma.reshape(1, DIM), beta.reshape(1, DIM))


def kernel(efeat, nfeat, src, dst, W_e, W_s, W_d, b1, W_out, b_out, gamma, beta):
    a, b = _node_proj(nfeat, W_s, W_d)
    gs, gd = _edge_gather(a, b, src, dst)
    out = _edge_mlp(efeat, gs, gd, W_e, b1, W_out, b_out, gamma, beta)
    return (out, nfeat)


# speedup vs baseline: 4.6008x; 1.2796x over previous
import functools

import jax
import jax.numpy as jnp
from jax import lax
from jax.experimental import pallas as pl
from jax.experimental.pallas import tpu as pltpu
from jax.experimental.pallas import tpu_sc as plsc

N_NODES = 10000
N_EDGES = 320000
DIM = 128

_NC = 2
_NS = 16
_NW = _NC * _NS
_EPW = N_EDGES // _NW
_CHUNK = 200
_NCHUNK = _EPW // _CHUNK
_NBUF = 3


def _proj_body(n_ref, ws_ref, wd_ref, a_ref, b_ref):
    n = n_ref[...]
    dn = (((1,), (1,)), ((), ()))
    a_ref[...] = lax.dot_general(n, ws_ref[...], dn,
                                 preferred_element_type=jnp.float32)
    b_ref[...] = lax.dot_general(n, wd_ref[...], dn,
                                 preferred_element_type=jnp.float32)


def _node_proj(nfeat, W_s, W_d):
    out_sd = jax.ShapeDtypeStruct((N_NODES, DIM), jnp.float32)
    return pl.pallas_call(
        _proj_body,
        out_shape=(out_sd, out_sd),
    )(nfeat, W_s, W_d)


def _gather_body(a_hbm, b_hbm, src_hbm, dst_hbm, gsum_hbm,
                 idx_s, idx_d, r0, r1, r2, s0, s1, s2):
    wid = lax.axis_index("s") * _NC + lax.axis_index("c")
    wbase = wid * _EPW
    bufs, sems = (r0, r1, r2), (s0, s1, s2)

    pltpu.sync_copy(src_hbm.at[pl.ds(wbase, _EPW)], idx_s)
    pltpu.sync_copy(dst_hbm.at[pl.ds(wbase, _EPW)], idx_d)

    def start_a(c, buf):
        off = c * _CHUNK
        pltpu.async_copy(
            a_hbm.at[idx_s.at[pl.ds(off, _CHUNK)]], bufs[buf], sems[buf])

    def start_b_add(c, buf):
        off = c * _CHUNK
        pltpu.async_copy(
            b_hbm.at[idx_d.at[pl.ds(off, _CHUNK)]], bufs[buf], sems[buf],
            add=True)

    def wait(buf):
        pltpu.make_async_copy(
            a_hbm.at[idx_s.at[pl.ds(0, _CHUNK)]], bufs[buf], sems[buf]).wait()

    start_a(0, 0)
    wait(0)
    start_b_add(0, 0)
    start_a(1, 1)

    @pl.loop(0, _NCHUNK, step=_NBUF)
    def _outer(j):
        for b in range(_NBUF):
            c = j + b
            nb = (b + 1) % _NBUF
            nb2 = (b + 2) % _NBUF

            @pl.when(c < _NCHUNK)
            def _chunk_c():
                wait(b)

                @pl.when(c + 1 < _NCHUNK)
                def _():
                    wait(nb)
                    start_b_add(c + 1, nb)

                @pl.when(c + 2 < _NCHUNK)
                def _():
                    start_a(c + 2, nb2)

                pltpu.sync_copy(bufs[b], gsum_hbm.at[pl.ds(wbase + c * _CHUNK,
                                                           _CHUNK)])


def _edge_gather(a, b, src, dst):
    out_sd = jax.ShapeDtypeStruct((N_EDGES, DIM), jnp.float32)
    mesh = plsc.VectorSubcoreMesh(core_axis_name="c", subcore_axis_name="s")
    f = functools.partial(
        pl.kernel,
        out_type=out_sd,
        mesh=mesh,
        scratch_types=[
            pltpu.VMEM((_EPW,), jnp.int32),
            pltpu.VMEM((_EPW,), jnp.int32),
            pltpu.VMEM((_CHUNK, DIM), jnp.float32),
            pltpu.VMEM((_CHUNK, DIM), jnp.float32),
            pltpu.VMEM((_CHUNK, DIM), jnp.float32),
            pltpu.SemaphoreType.DMA,
            pltpu.SemaphoreType.DMA,
            pltpu.SemaphoreType.DMA,
        ],
    )(_gather_body)
    return f(a, b, src, dst)


def _edge_body(e_ref, g_ref, we_ref, b1_ref, wo_ref, bo_ref,
               gm_ref, bt_ref, out_ref):
    e = e_ref[...]
    dn = (((1,), (1,)), ((), ()))
    h = lax.dot_general(e, we_ref[...], dn, preferred_element_type=jnp.float32)
    h = h + g_ref[...] + b1_ref[...]
    h = h * (1.0 / (1.0 + jnp.exp(-h)))
    o = lax.dot_general(h, wo_ref[...], dn, preferred_element_type=jnp.float32)
    o = o + bo_ref[...]
    mean = jnp.mean(o, axis=-1, keepdims=True)
    cen = o - mean
    var = jnp.mean(cen * cen, axis=-1, keepdims=True)
    o = cen * lax.rsqrt(var + 1e-5) * gm_ref[...] + bt_ref[...]
    out_ref[...] = o + e


def _edge_mlp(efeat, gsum, W_e, b1, W_out, b_out, gamma, beta):
    blk = 2560
    grid = (N_EDGES // blk,)
    row_spec = pl.BlockSpec((blk, DIM), lambda i: (i, 0))
    w_spec = pl.BlockSpec((DIM, DIM), lambda i: (0, 0))
    v_spec = pl.BlockSpec((1, DIM), lambda i: (0, 0))
    return pl.pallas_call(
        _edge_body,
        grid=grid,
        in_specs=[row_spec, row_spec, w_spec, v_spec, w_spec,
                  v_spec, v_spec, v_spec],
        out_specs=row_spec,
        out_shape=jax.ShapeDtypeStruct((N_EDGES, DIM), jnp.float32),
    )(efeat, gsum, W_e, b1.reshape(1, DIM), W_out, b_out.reshape(1, DIM),
      gamma.reshape(1, DIM), beta.reshape(1, DIM))


def kernel(efeat, nfeat, src, dst, W_e, W_s, W_d, b1, W_out, b_out, gamma, beta):
    a, b = _node_proj(nfeat, W_s, W_d)
    gsum = _edge_gather(a, b, src, dst)
    out = _edge_mlp(efeat, gsum, W_e, b1, W_out, b_out, gamma, beta)
    return (out, nfeat)


# speedup vs baseline: 4.8923x; 1.0634x over previous
import functools

import jax
import jax.numpy as jnp
from jax import lax
from jax.experimental import pallas as pl
from jax.experimental.pallas import tpu as pltpu
from jax.experimental.pallas import tpu_sc as plsc

N_NODES = 10000
N_EDGES = 320000
DIM = 128

_NC = 2
_NS = 16
_NW = _NC * _NS
_NSEG = 5
_ESEG = N_EDGES // _NSEG
_EPW = _ESEG // _NW
_CHUNK = 200
_NCHUNK = _EPW // _CHUNK
_NBUF = 3
_BLK = 2000
_SSTEP = _ESEG // _BLK


def _proj_body(n_ref, ws_ref, wd_ref, a_ref, b_ref):
    n = n_ref[...]
    dn = (((1,), (1,)), ((), ()))
    a_ref[...] = lax.dot_general(n, ws_ref[...], dn,
                                 preferred_element_type=jnp.float32)
    b_ref[...] = lax.dot_general(n, wd_ref[...], dn,
                                 preferred_element_type=jnp.float32)


def _node_proj(nfeat, W_s, W_d):
    out_sd = jax.ShapeDtypeStruct((N_NODES, DIM), jnp.float32)
    return pl.pallas_call(
        _proj_body,
        out_shape=(out_sd, out_sd),
    )(nfeat, W_s, W_d)


def _gather_body(seg, a_hbm, b_hbm, src_hbm, dst_hbm, gsum_hbm,
                 idx_s, idx_d, r0, r1, r2, s0, s1, s2):
    wid = lax.axis_index("s") * _NC + lax.axis_index("c")
    wbase = seg * _ESEG + wid * _EPW
    bufs, sems = (r0, r1, r2), (s0, s1, s2)

    pltpu.sync_copy(src_hbm.at[pl.ds(wbase, _EPW)], idx_s)
    pltpu.sync_copy(dst_hbm.at[pl.ds(wbase, _EPW)], idx_d)

    def start_a(c, buf):
        off = c * _CHUNK
        pltpu.async_copy(
            a_hbm.at[idx_s.at[pl.ds(off, _CHUNK)]], bufs[buf], sems[buf])

    def start_b_add(c, buf):
        off = c * _CHUNK
        pltpu.async_copy(
            b_hbm.at[idx_d.at[pl.ds(off, _CHUNK)]], bufs[buf], sems[buf],
            add=True)

    def wait(buf):
        pltpu.make_async_copy(
            a_hbm.at[idx_s.at[pl.ds(0, _CHUNK)]], bufs[buf], sems[buf]).wait()

    start_a(0, 0)
    wait(0)
    start_b_add(0, 0)
    start_a(1, 1)

    @pl.loop(0, _NCHUNK, step=_NBUF)
    def _outer(j):
        for b in range(_NBUF):
            c = j + b
            nb = (b + 1) % _NBUF
            nb2 = (b + 2) % _NBUF

            @pl.when(c < _NCHUNK)
            def _chunk_c():
                wait(b)

                @pl.when(c + 1 < _NCHUNK)
                def _():
                    wait(nb)
                    start_b_add(c + 1, nb)

                @pl.when(c + 2 < _NCHUNK)
                def _():
                    start_a(c + 2, nb2)

                pltpu.sync_copy(
                    bufs[b],
                    gsum_hbm.at[pl.ds(wid * _EPW + c * _CHUNK, _CHUNK)])


def _edge_gather(a, b, src, dst, seg):
    out_sd = jax.ShapeDtypeStruct((_ESEG, DIM), jnp.float32)
    mesh = plsc.VectorSubcoreMesh(core_axis_name="c", subcore_axis_name="s")
    f = functools.partial(
        pl.kernel,
        out_type=out_sd,
        mesh=mesh,
        scratch_types=[
            pltpu.VMEM((_EPW,), jnp.int32),
            pltpu.VMEM((_EPW,), jnp.int32),
            pltpu.VMEM((_CHUNK, DIM), jnp.float32),
            pltpu.VMEM((_CHUNK, DIM), jnp.float32),
            pltpu.VMEM((_CHUNK, DIM), jnp.float32),
            pltpu.SemaphoreType.DMA,
            pltpu.SemaphoreType.DMA,
            pltpu.SemaphoreType.DMA,
        ],
    )(functools.partial(_gather_body, seg))
    return f(a, b, src, dst)


def _edge_mlp_body(e_ref, g_ref, we_ref, b1_ref, wo_ref, bo_ref,
                   gm_ref, bt_ref, out_ref):
    e = e_ref[...]
    dn = (((1,), (1,)), ((), ()))
    h = lax.dot_general(e, we_ref[...], dn, preferred_element_type=jnp.float32)
    h = h + g_ref[...] + b1_ref[...]
    h = h * (1.0 / (1.0 + jnp.exp(-h)))
    o = lax.dot_general(h, wo_ref[...], dn, preferred_element_type=jnp.float32)
    o = o + bo_ref[...]
    mean = jnp.mean(o, axis=-1, keepdims=True)
    cen = o - mean
    var = jnp.mean(cen * cen, axis=-1, keepdims=True)
    o = cen * lax.rsqrt(var + 1e-5) * gm_ref[...] + bt_ref[...]
    out_ref[...] = o + e


def _edge_mlp_seg_first(e_ref, g_ref, *rest):
    _edge_mlp_body(e_ref, g_ref, *rest)


def _edge_mlp_seg_chain(_buf_ref, e_ref, g_ref, *rest):
    _edge_mlp_body(e_ref, g_ref, *rest)


def _edge_mlp(prev, efeat, gsum_seg, seg, W_e, b1, W_out, b_out, gamma, beta):
    seg_row = pl.BlockSpec((_BLK, DIM), lambda i, s=seg: (s * _SSTEP + i, 0))
    loc_row = pl.BlockSpec((_BLK, DIM), lambda i: (i, 0))
    w_spec = pl.BlockSpec((DIM, DIM), lambda i: (0, 0))
    v_spec = pl.BlockSpec((1, DIM), lambda i: (0, 0))
    any_spec = pl.BlockSpec(memory_space=pl.ANY)
    common = [loc_row, w_spec, v_spec, w_spec, v_spec, v_spec, v_spec]
    args = (efeat, gsum_seg, W_e, b1.reshape(1, DIM), W_out,
            b_out.reshape(1, DIM), gamma.reshape(1, DIM), beta.reshape(1, DIM))
    if prev is None:
        return pl.pallas_call(
            _edge_mlp_seg_first,
            grid=(_SSTEP,),
            in_specs=[seg_row] + common,
            out_specs=seg_row,
            out_shape=jax.ShapeDtypeStruct((N_EDGES, DIM), jnp.float32),
        )(*args)
    return pl.pallas_call(
        _edge_mlp_seg_chain,
        grid=(_SSTEP,),
        in_specs=[any_spec, seg_row] + common,
        out_specs=seg_row,
        out_shape=jax.ShapeDtypeStruct((N_EDGES, DIM), jnp.float32),
        input_output_aliases={0: 0},
    )(prev, *args)


def kernel(efeat, nfeat, src, dst, W_e, W_s, W_d, b1, W_out, b_out, gamma, beta):
    a, b = _node_proj(nfeat, W_s, W_d)
    gsums = [_edge_gather(a, b, src, dst, seg) for seg in range(_NSEG)]
    out = None
    for seg in range(_NSEG):
        out = _edge_mlp(out, efeat, gsums[seg], seg,
                        W_e, b1, W_out, b_out, gamma, beta)
    return (out, nfeat)
